# baseline (speedup 1.0000x reference)
import functools

import jax
import jax.numpy as jnp
import numpy as np
from jax import lax
from jax.experimental import pallas as pl
from jax.experimental.pallas import tpu as pltpu
from jax.experimental.pallas import tpu_sc as plsc

STEP = 16000
N_FRAMES = 128
CHANNELS = 2
ROWS = CHANNELS * N_FRAMES

with jax.default_device(jax.local_devices(backend="cpu")[0]):
    _PERM = np.asarray(jax.random.permutation(jax.random.key(1), N_FRAMES))
_ROW_IDX = np.concatenate(
    [c * N_FRAMES + _PERM for c in range(CHANNELS)]
).astype(np.int32)

_NC = 2
_NS = 16
_NW = _NC * _NS
_RPW = ROWS // _NW

_mesh = plsc.VectorSubcoreMesh(core_axis_name="c", subcore_axis_name="s")


@functools.partial(
    pl.kernel,
    mesh=_mesh,
    out_type=jax.ShapeDtypeStruct((ROWS, STEP), jnp.float32),
    scratch_types=[
        pltpu.VMEM((_RPW,), jnp.int32),
        pltpu.VMEM((_RPW, STEP), jnp.float32),
        pltpu.SemaphoreType.DMA,
    ],
)
def _shuffle_rows(src_hbm, idx_hbm, out_hbm, idx_v, rows_v, sem):
    wid = lax.axis_index("s") * _NC + lax.axis_index("c")
    base = wid * _RPW
    pltpu.sync_copy(idx_hbm.at[pl.ds(base, _RPW)], idx_v)
    pltpu.async_copy(src_hbm.at[idx_v], rows_v, sem).wait()
    pltpu.sync_copy(rows_v, out_hbm.at[pl.ds(base, _RPW)])


def kernel(waveform):
    frames = waveform.reshape(ROWS, STEP)
    out = _shuffle_rows(frames, jnp.asarray(_ROW_IDX))
    return out.reshape(CHANNELS, N_FRAMES * STEP)
